# baseline (speedup 1.0000x reference)
import functools

import jax
import jax.numpy as jnp
from jax import lax
from jax.experimental import pallas as pl
from jax.experimental.pallas import tpu as pltpu
from jax.experimental.pallas import tpu_sc as plsc

N_ATOMS = 10000
N_BONDS = 320000
MAX_NB = 32
BOND_FDIM = 16
EMB = 128
NUM_LAYER = 3
NCG = EMB // 16

NC = 2
NS = 16
NW = NC * NS

NA_PAD = 10240
NB_PAD = 327680
ATOMS_PER_W = NA_PAD // NW
BONDS_PER_W = NB_PAD // NW

CA = 4
CB = 128
A_NCH = ATOMS_PER_W // CA
B_NCH = BONDS_PER_W // CB

A_NBUF = 4
B_NBUF = 2

_mesh = plsc.VectorSubcoreMesh(
    core_axis_name="c", subcore_axis_name="s", num_cores=NC, num_subcores=NS)


def _worker_id():
  return lax.axis_index("c") * NS + lax.axis_index("s")


@functools.partial(
    pl.kernel,
    out_type=jax.ShapeDtypeStruct((NA_PAD, EMB), jnp.float32),
    mesh=_mesh,
    scratch_types=(
        [pltpu.VMEM((ATOMS_PER_W * MAX_NB,), jnp.int32),
         pltpu.VMEM((ATOMS_PER_W, EMB), jnp.float32)]
        + [pltpu.VMEM((CA * MAX_NB, EMB), jnp.float32) for _ in range(A_NBUF)]
        + [pltpu.SemaphoreType.DMA for _ in range(A_NBUF)]
    ),
)
def _sc_seg_sum(msg_hbm, a2b_hbm, out_hbm, idx_v, acc_v, *bufs_and_sems):
  bufs = bufs_and_sems[:A_NBUF]
  sems = bufs_and_sems[A_NBUF:]
  wid = _worker_id()
  atom_base = wid * ATOMS_PER_W

  pltpu.sync_copy(a2b_hbm.at[pl.ds(atom_base * MAX_NB, ATOMS_PER_W * MAX_NB)],
                  idx_v)

  def gather(ch, b):
    pltpu.async_copy(
        msg_hbm.at[idx_v.at[pl.ds(ch * 128, 128)]], bufs[b], sems[b])

  def gather_wait(ch, b):
    pltpu.make_async_copy(
        msg_hbm.at[idx_v.at[pl.ds(ch * 128, 128)]], bufs[b], sems[b]).wait()

  def compute(ch, b):
    buf = bufs[b]

    @pl.loop(0, CA)
    def _atom(a):
      row0 = a * MAX_NB
      for cg in range(NCG):
        col = pl.ds(cg * 16, 16)
        acc = buf[row0, col]
        for r in range(1, MAX_NB):
          acc = acc + buf[row0 + r, col]
        acc_v[ch * CA + a, col] = acc

  for b in range(A_NBUF):
    gather(b, b)

  @pl.loop(0, A_NCH - A_NBUF, step=A_NBUF)
  def _main(c):
    for b in range(A_NBUF):
      ch = c + b
      gather_wait(ch, b)
      compute(ch, b)
      gather(ch + A_NBUF, b)

  for b in range(A_NBUF):
    ch = A_NCH - A_NBUF + b
    gather_wait(ch, b)
    compute(ch, b)

  pltpu.sync_copy(acc_v, out_hbm.at[pl.ds(atom_base, ATOMS_PER_W)])


@functools.partial(
    pl.kernel,
    out_type=jax.ShapeDtypeStruct((NB_PAD, EMB), jnp.float32),
    mesh=_mesh,
    scratch_types=(
        [pltpu.VMEM((BONDS_PER_W,), jnp.int32),
         pltpu.VMEM((BONDS_PER_W,), jnp.int32)]
        + [pltpu.VMEM((CB, EMB), jnp.float32) for _ in range(3 * B_NBUF)]
        + [pltpu.SemaphoreType.DMA for _ in range(3 * B_NBUF)]
    ),
)
def _sc_bond_msg(a_hbm, msg_hbm, b2a_hbm, b2revb_hbm, out_hbm,
                 ia_v, ir_v, *bufs_and_sems):
  bufa = bufs_and_sems[0:B_NBUF]
  bufr = bufs_and_sems[B_NBUF:2 * B_NBUF]
  outv = bufs_and_sems[2 * B_NBUF:3 * B_NBUF]
  sema = bufs_and_sems[3 * B_NBUF:4 * B_NBUF]
  semr = bufs_and_sems[4 * B_NBUF:5 * B_NBUF]
  semw = bufs_and_sems[5 * B_NBUF:6 * B_NBUF]
  wid = _worker_id()
  base = wid * BONDS_PER_W

  pltpu.sync_copy(b2a_hbm.at[pl.ds(base, BONDS_PER_W)], ia_v)
  pltpu.sync_copy(b2revb_hbm.at[pl.ds(base, BONDS_PER_W)], ir_v)

  def gathers(ch, b):
    pltpu.async_copy(
        a_hbm.at[ia_v.at[pl.ds(ch * CB, CB)]], bufa[b], sema[b])
    pltpu.async_copy(
        msg_hbm.at[ir_v.at[pl.ds(ch * CB, CB)]], bufr[b], semr[b])

  def gathers_wait(ch, b):
    pltpu.make_async_copy(
        a_hbm.at[ia_v.at[pl.ds(ch * CB, CB)]], bufa[b], sema[b]).wait()
    pltpu.make_async_copy(
        msg_hbm.at[ir_v.at[pl.ds(ch * CB, CB)]], bufr[b], semr[b]).wait()

  def compute(b):
    @pl.loop(0, CB)
    def _row(r):
      for cg in range(NCG):
        col = pl.ds(cg * 16, 16)
        outv[b][r, col] = bufa[b][r, col] - bufr[b][r, col]

  def write(ch, b):
    pltpu.async_copy(
        outv[b], out_hbm.at[pl.ds(base + ch * CB, CB)], semw[b])

  def write_wait(ch, b):
    pltpu.make_async_copy(
        outv[b], out_hbm.at[pl.ds(base + ch * CB, CB)], semw[b]).wait()

  for b in range(B_NBUF):
    gathers(b, b)
  for b in range(B_NBUF):
    gathers_wait(b, b)
    compute(b)
    write(b, b)
    gathers(b + B_NBUF, b)

  @pl.loop(0, B_NCH - 2 * B_NBUF, step=B_NBUF)
  def _main(c):
    for b in range(B_NBUF):
      ch = c + b + B_NBUF
      gathers_wait(ch, b)
      write_wait(ch - B_NBUF, b)
      compute(b)
      write(ch, b)
      gathers(ch + B_NBUF, b)

  for b in range(B_NBUF):
    ch = B_NCH - B_NBUF + b
    gathers_wait(ch, b)
    write_wait(ch - B_NBUF, b)
    compute(b)
    write(ch, b)
  for b in range(B_NBUF):
    write_wait(B_NCH - B_NBUF + b, b)


TB1 = 4000
TBO = 2000


def _tc_embed_body(fb_ref, wi_ref, bi_ref, msg_ref):
  inp = lax.dot_general(fb_ref[...], wi_ref[...], (((1,), (1,)), ((), ())),
                        preferred_element_type=jnp.float32) + bi_ref[...]
  msg_ref[...] = jnp.maximum(inp, 0.0)


def _tc_embed(f_bonds, W_i, b_i):
  return pl.pallas_call(
      _tc_embed_body,
      grid=(N_BONDS // TB1,),
      in_specs=[
          pl.BlockSpec((TB1, BOND_FDIM), lambda i: (i, 0)),
          pl.BlockSpec((EMB, BOND_FDIM), lambda i: (0, 0)),
          pl.BlockSpec((1, EMB), lambda i: (0, 0)),
      ],
      out_specs=pl.BlockSpec((TB1, EMB), lambda i: (i, 0)),
      out_shape=jax.ShapeDtypeStruct((N_BONDS, EMB), jnp.float32),
      compiler_params=pltpu.CompilerParams(
          dimension_semantics=("arbitrary",)),
  )(f_bonds, W_i, b_i.reshape(1, EMB))


def _tc_update_body(m_ref, fb_ref, wh_ref, bh_ref, wi_ref, bi_ref, out_ref):
  inp = lax.dot_general(fb_ref[...], wi_ref[...], (((1,), (1,)), ((), ())),
                        preferred_element_type=jnp.float32) + bi_ref[...]
  h = lax.dot_general(m_ref[...], wh_ref[...], (((1,), (1,)), ((), ())),
                      preferred_element_type=jnp.float32) + bh_ref[...]
  out_ref[...] = jnp.maximum(inp + h, 0.0)


def _tc_update(m_pad, f_bonds, W_h, b_h, W_i, b_i):
  return pl.pallas_call(
      _tc_update_body,
      grid=(N_BONDS // TB1,),
      in_specs=[
          pl.BlockSpec((TB1, EMB), lambda i: (i, 0)),
          pl.BlockSpec((TB1, BOND_FDIM), lambda i: (i, 0)),
          pl.BlockSpec((EMB, EMB), lambda i: (0, 0)),
          pl.BlockSpec((1, EMB), lambda i: (0, 0)),
          pl.BlockSpec((EMB, BOND_FDIM), lambda i: (0, 0)),
          pl.BlockSpec((1, EMB), lambda i: (0, 0)),
      ],
      out_specs=pl.BlockSpec((TB1, EMB), lambda i: (i, 0)),
      out_shape=jax.ShapeDtypeStruct((N_BONDS, EMB), jnp.float32),
      compiler_params=pltpu.CompilerParams(
          dimension_semantics=("arbitrary",)),
  )(m_pad, f_bonds, W_h, b_h.reshape(1, EMB), W_i, b_i.reshape(1, EMB))


def _tc_head_body(fa_ref, am_ref, wo1_ref, wo2_ref, bo_ref, out_ref):
  h = lax.dot_general(fa_ref[...], wo1_ref[...], (((1,), (1,)), ((), ())),
                      preferred_element_type=jnp.float32)
  h = h + lax.dot_general(am_ref[...], wo2_ref[...], (((1,), (1,)), ((), ())),
                          preferred_element_type=jnp.float32)
  out_ref[...] = jnp.maximum(h + bo_ref[...], 0.0)


def _tc_head(f_atoms, a_msg_pad, W_o, b_o):
  return pl.pallas_call(
      _tc_head_body,
      grid=(N_ATOMS // TBO,),
      in_specs=[
          pl.BlockSpec((TBO, EMB), lambda i: (i, 0)),
          pl.BlockSpec((TBO, EMB), lambda i: (i, 0)),
          pl.BlockSpec((EMB, EMB), lambda i: (0, 0)),
          pl.BlockSpec((EMB, EMB), lambda i: (0, 0)),
          pl.BlockSpec((1, EMB), lambda i: (0, 0)),
      ],
      out_specs=pl.BlockSpec((TBO, EMB), lambda i: (i, 0)),
      out_shape=jax.ShapeDtypeStruct((N_ATOMS, EMB), jnp.float32),
      compiler_params=pltpu.CompilerParams(
          dimension_semantics=("arbitrary",)),
  )(f_atoms, a_msg_pad, W_o[:, :EMB], W_o[:, EMB:], b_o.reshape(1, EMB))


def kernel(f_atoms, f_bonds, a2b, b2a, b2revb, W_i, b_i, W_h, b_h, W_o, b_o):
  a2b = a2b.astype(jnp.int32)
  b2a = b2a.astype(jnp.int32)
  b2revb = b2revb.astype(jnp.int32)
  a2b_flat = jnp.concatenate(
      [a2b, jnp.zeros((NA_PAD - N_ATOMS, MAX_NB), jnp.int32)], axis=0
  ).reshape(-1)
  pad_b = jnp.zeros((NB_PAD - N_BONDS,), jnp.int32)
  b2a_pad = jnp.concatenate([b2a, pad_b])
  b2revb_pad = jnp.concatenate([b2revb, pad_b])

  msg = _tc_embed(f_bonds, W_i, b_i)
  for _ in range(NUM_LAYER - 1):
    a_msg = _sc_seg_sum(msg, a2b_flat)
    m_pad = _sc_bond_msg(a_msg, msg, b2a_pad, b2revb_pad)
    msg = _tc_update(m_pad, f_bonds, W_h, b_h, W_i, b_i)
  a_msg = _sc_seg_sum(msg, a2b_flat)
  return _tc_head(f_atoms, a_msg, W_o, b_o)


# speedup vs baseline: 1.0633x; 1.0633x over previous
import functools

import jax
import jax.numpy as jnp
from jax import lax
from jax.experimental import pallas as pl
from jax.experimental.pallas import tpu as pltpu
from jax.experimental.pallas import tpu_sc as plsc

N_ATOMS = 10000
N_BONDS = 320000
MAX_NB = 32
BOND_FDIM = 16
EMB = 128
NUM_LAYER = 3
NCG = EMB // 16

NC = 2
NS = 16
NW = NC * NS

NA_PAD = 10240
NB_PAD = 327680
ATOMS_PER_W = NA_PAD // NW
BONDS_PER_W = NB_PAD // NW

CA = 4
A_ROWS = CA * MAX_NB
A_NCH = ATOMS_PER_W // CA
A_NBUF = 4

CB = 128
B_NCH = BONDS_PER_W // CB
B_NBUF = 2

_mesh = plsc.VectorSubcoreMesh(
    core_axis_name="c", subcore_axis_name="s", num_cores=NC, num_subcores=NS)


def _worker_id():
  return lax.axis_index("c") * NS + lax.axis_index("s")


@functools.partial(
    pl.kernel,
    out_type=jax.ShapeDtypeStruct((NA_PAD, EMB), jnp.float32),
    mesh=_mesh,
    scratch_types=(
        [pltpu.VMEM((ATOMS_PER_W * MAX_NB,), jnp.int32),
         pltpu.VMEM((ATOMS_PER_W, EMB), jnp.float32)]
        + [pltpu.VMEM((A_ROWS, EMB), jnp.float32) for _ in range(A_NBUF)]
        + [pltpu.SemaphoreType.DMA for _ in range(A_NBUF)]
    ),
)
def _sc_seg_sum(msg_hbm, a2b_hbm, out_hbm, idx_v, acc_v, *bufs_and_sems):
  bufs = bufs_and_sems[:A_NBUF]
  sems = bufs_and_sems[A_NBUF:]
  wid = _worker_id()
  atom_base = wid * ATOMS_PER_W

  pltpu.sync_copy(a2b_hbm.at[pl.ds(atom_base * MAX_NB, ATOMS_PER_W * MAX_NB)],
                  idx_v)

  def gather(ch, b):
    return pltpu.async_copy(
        msg_hbm.at[idx_v.at[pl.ds(ch * A_ROWS, A_ROWS)]], bufs[b], sems[b])

  def gather_wait(ch, b):
    pltpu.make_async_copy(
        msg_hbm.at[idx_v.at[pl.ds(ch * A_ROWS, A_ROWS)]], bufs[b],
        sems[b]).wait()

  def compute(ch, b):
    buf = bufs[b]

    @pl.loop(0, CA)
    def _atom(a):
      row0 = a * MAX_NB
      for cg in range(NCG):
        col = pl.ds(cg * 16, 16)
        acc = buf[row0, col]
        for r in range(1, MAX_NB):
          acc = acc + buf[row0 + r, col]
        acc_v[ch * CA + a, col] = acc

  for b in range(A_NBUF):
    gather(b, b)

  @pl.loop(0, A_NCH - A_NBUF, step=A_NBUF)
  def _main(c):
    for b in range(A_NBUF):
      ch = c + b
      gather_wait(ch, b)
      compute(ch, b)
      gather(ch + A_NBUF, b)

  for b in range(A_NBUF):
    ch = A_NCH - A_NBUF + b
    gather_wait(ch, b)
    compute(ch, b)

  pltpu.sync_copy(acc_v, out_hbm.at[pl.ds(atom_base, ATOMS_PER_W)])


@functools.partial(
    pl.kernel,
    out_type=jax.ShapeDtypeStruct((NB_PAD, EMB), jnp.float32),
    mesh=_mesh,
    scratch_types=(
        [pltpu.VMEM((BONDS_PER_W,), jnp.int32),
         pltpu.VMEM((BONDS_PER_W,), jnp.int32)]
        + [pltpu.VMEM((CB, EMB), jnp.float32) for _ in range(3 * B_NBUF)]
        + [pltpu.SemaphoreType.DMA for _ in range(3 * B_NBUF)]
    ),
)
def _sc_bond_msg(a_hbm, msg_hbm, b2a_hbm, b2revb_hbm, out_hbm,
                 ia_v, ir_v, *bufs_and_sems):
  bufa = bufs_and_sems[0:B_NBUF]
  bufr = bufs_and_sems[B_NBUF:2 * B_NBUF]
  outv = bufs_and_sems[2 * B_NBUF:3 * B_NBUF]
  sema = bufs_and_sems[3 * B_NBUF:4 * B_NBUF]
  semr = bufs_and_sems[4 * B_NBUF:5 * B_NBUF]
  semw = bufs_and_sems[5 * B_NBUF:6 * B_NBUF]
  wid = _worker_id()
  base = wid * BONDS_PER_W

  pltpu.sync_copy(b2a_hbm.at[pl.ds(base, BONDS_PER_W)], ia_v)
  pltpu.sync_copy(b2revb_hbm.at[pl.ds(base, BONDS_PER_W)], ir_v)

  def gathers(ch, b):
    pltpu.async_copy(a_hbm.at[ia_v.at[pl.ds(ch * CB, CB)]], bufa[b], sema[b])
    pltpu.async_copy(msg_hbm.at[ir_v.at[pl.ds(ch * CB, CB)]], bufr[b], semr[b])

  def gathers_wait(ch, b):
    pltpu.make_async_copy(
        a_hbm.at[ia_v.at[pl.ds(ch * CB, CB)]], bufa[b], sema[b]).wait()
    pltpu.make_async_copy(
        msg_hbm.at[ir_v.at[pl.ds(ch * CB, CB)]], bufr[b], semr[b]).wait()

  def compute(b):
    @pl.loop(0, CB)
    def _row(r):
      for cg in range(NCG):
        col = pl.ds(cg * 16, 16)
        outv[b][r, col] = bufa[b][r, col] - bufr[b][r, col]

  def write(ch, b):
    return pltpu.async_copy(
        outv[b], out_hbm.at[pl.ds(base + ch * CB, CB)], semw[b])

  def write_wait(ch, b):
    pltpu.make_async_copy(
        outv[b], out_hbm.at[pl.ds(base + ch * CB, CB)], semw[b]).wait()

  for b in range(B_NBUF):
    gathers(b, b)
  for b in range(B_NBUF):
    gathers_wait(b, b)
    compute(b)
    write(b, b)
    gathers(b + B_NBUF, b)

  @pl.loop(0, B_NCH - 2 * B_NBUF, step=B_NBUF)
  def _main(c):
    for b in range(B_NBUF):
      ch = c + b + B_NBUF
      gathers_wait(ch, b)
      write_wait(ch - B_NBUF, b)
      compute(b)
      write(ch, b)
      gathers(ch + B_NBUF, b)

  for b in range(B_NBUF):
    ch = B_NCH - B_NBUF + b
    gathers_wait(ch, b)
    write_wait(ch - B_NBUF, b)
    compute(b)
    write(ch, b)
  for b in range(B_NBUF):
    write_wait(B_NCH - B_NBUF + b, b)


TB1 = 4000
TBO = 2000


def _tc_embed_body(fb_ref, wi_ref, bi_ref, inp_ref, msg_ref):
  inp = lax.dot_general(fb_ref[...], wi_ref[...], (((1,), (1,)), ((), ())),
                        preferred_element_type=jnp.float32) + bi_ref[...]
  inp_ref[...] = inp
  msg_ref[...] = jnp.maximum(inp, 0.0)


def _tc_embed(f_bonds, W_i, b_i):
  return pl.pallas_call(
      _tc_embed_body,
      grid=(N_BONDS // TB1,),
      in_specs=[
          pl.BlockSpec((TB1, BOND_FDIM), lambda i: (i, 0)),
          pl.BlockSpec((EMB, BOND_FDIM), lambda i: (0, 0)),
          pl.BlockSpec((1, EMB), lambda i: (0, 0)),
      ],
      out_specs=[
          pl.BlockSpec((TB1, EMB), lambda i: (i, 0)),
          pl.BlockSpec((TB1, EMB), lambda i: (i, 0)),
      ],
      out_shape=[
          jax.ShapeDtypeStruct((N_BONDS, EMB), jnp.float32),
          jax.ShapeDtypeStruct((N_BONDS, EMB), jnp.float32),
      ],
      compiler_params=pltpu.CompilerParams(
          dimension_semantics=("arbitrary",)),
  )(f_bonds, W_i, b_i.reshape(1, EMB))


def _tc_update_body(m_ref, inp_ref, wh_ref, bh_ref, out_ref):
  h = lax.dot_general(m_ref[...], wh_ref[...], (((1,), (1,)), ((), ())),
                      preferred_element_type=jnp.float32) + bh_ref[...]
  out_ref[...] = jnp.maximum(inp_ref[...] + h, 0.0)


def _tc_update(m_pad, inp, W_h, b_h):
  return pl.pallas_call(
      _tc_update_body,
      grid=(N_BONDS // TB1,),
      in_specs=[
          pl.BlockSpec((TB1, EMB), lambda i: (i, 0)),
          pl.BlockSpec((TB1, EMB), lambda i: (i, 0)),
          pl.BlockSpec((EMB, EMB), lambda i: (0, 0)),
          pl.BlockSpec((1, EMB), lambda i: (0, 0)),
      ],
      out_specs=pl.BlockSpec((TB1, EMB), lambda i: (i, 0)),
      out_shape=jax.ShapeDtypeStruct((N_BONDS, EMB), jnp.float32),
      compiler_params=pltpu.CompilerParams(
          dimension_semantics=("arbitrary",)),
  )(m_pad, inp, W_h, b_h.reshape(1, EMB))


def _tc_head_body(fa_ref, am_ref, wo1_ref, wo2_ref, bo_ref, out_ref):
  h = lax.dot_general(fa_ref[...], wo1_ref[...], (((1,), (1,)), ((), ())),
                      preferred_element_type=jnp.float32)
  h = h + lax.dot_general(am_ref[...], wo2_ref[...], (((1,), (1,)), ((), ())),
                          preferred_element_type=jnp.float32)
  out_ref[...] = jnp.maximum(h + bo_ref[...], 0.0)


def _tc_head(f_atoms, a_msg_pad, W_o, b_o):
  return pl.pallas_call(
      _tc_head_body,
      grid=(N_ATOMS // TBO,),
      in_specs=[
          pl.BlockSpec((TBO, EMB), lambda i: (i, 0)),
          pl.BlockSpec((TBO, EMB), lambda i: (i, 0)),
          pl.BlockSpec((EMB, EMB), lambda i: (0, 0)),
          pl.BlockSpec((EMB, EMB), lambda i: (0, 0)),
          pl.BlockSpec((1, EMB), lambda i: (0, 0)),
      ],
      out_specs=pl.BlockSpec((TBO, EMB), lambda i: (i, 0)),
      out_shape=jax.ShapeDtypeStruct((N_ATOMS, EMB), jnp.float32),
      compiler_params=pltpu.CompilerParams(
          dimension_semantics=("arbitrary",)),
  )(f_atoms, a_msg_pad, W_o[:, :EMB], W_o[:, EMB:], b_o.reshape(1, EMB))


def kernel(f_atoms, f_bonds, a2b, b2a, b2revb, W_i, b_i, W_h, b_h, W_o, b_o):
  a2b = a2b.astype(jnp.int32)
  b2a = b2a.astype(jnp.int32)
  b2revb = b2revb.astype(jnp.int32)
  a2b_flat = jnp.concatenate(
      [a2b, jnp.zeros((NA_PAD - N_ATOMS, MAX_NB), jnp.int32)], axis=0
  ).reshape(-1)
  pad_b = jnp.zeros((NB_PAD - N_BONDS,), jnp.int32)
  b2a_pad = jnp.concatenate([b2a, pad_b])
  b2revb_pad = jnp.concatenate([b2revb, pad_b])

  inp, msg = _tc_embed(f_bonds, W_i, b_i)
  for _ in range(NUM_LAYER - 1):
    a_msg = _sc_seg_sum(msg, a2b_flat)
    m_pad = _sc_bond_msg(a_msg, msg, b2a_pad, b2revb_pad)
    msg = _tc_update(m_pad, inp, W_h, b_h)
  a_msg = _sc_seg_sum(msg, a2b_flat)
  return _tc_head(f_atoms, a_msg, W_o, b_o)
